# baseline (speedup 1.0000x reference)
import jax
import jax.numpy as jnp
from jax import lax
from jax.experimental import pallas as pl
from jax.experimental.pallas import tpu as pltpu
from jax.experimental.pallas import tpu_sc as plsc

NC, NS, LANES = 2, 16, 16

GRID = 200
R_GRID = GRID * GRID
R_TOT = R_GRID + NS
ROWS_Z = R_TOT // NS
ROWS_R = R_GRID // NS
CH = 512
CHALF = 32


def _sc_scatter(xr, gt):
    B, Np, C = xr.shape
    nchunk = Np // CH
    assert Np % CH == 0 and C == 2 * CHALF
    kmax = (nchunk + NS - 1) // NS
    mesh = plsc.VectorSubcoreMesh(
        core_axis_name="c", subcore_axis_name="s",
        num_cores=NC, num_subcores=NS)

    def body(x_hbm, g_hbm, out_hbm, geom_v, xbuf, idxbuf, zb, acc):
        core = lax.axis_index("c")
        tid = lax.axis_index("s")
        ch0 = core * CHALF

        def zb_init(i, carry):
            zb[i, pl.ds(0, LANES)] = jnp.zeros((LANES,), jnp.float32)
            zb[i, pl.ds(LANES, LANES)] = jnp.zeros((LANES,), jnp.float32)
            return carry
        lax.fori_loop(0, zb.shape[0], zb_init, 0)

        for b in range(B):
            r0 = tid * ROWS_Z
            off, rem = 0, ROWS_Z
            while rem > 0:
                n = min(rem, zb.shape[0])
                pltpu.sync_copy(zb.at[pl.ds(0, n)], acc.at[pl.ds(r0 + off, n)])
                off += n
                rem -= n
            plsc.subcore_barrier()

            def chunk_body(k, carry):
                c = k * NS + tid

                @pl.when(c < nchunk)
                def _():
                    base = c * CH
                    pltpu.sync_copy(
                        g_hbm.at[b, pl.ds(base * 3, CH * 3)], geom_v)
                    pltpu.sync_copy(
                        x_hbm.at[b, pl.ds(base, CH), pl.ds(ch0, CHALF)], xbuf)
                    lane3 = lax.iota(jnp.int32, LANES) * 3
                    for l in range(CH // LANES):
                        s = l * LANES * 3
                        gx = plsc.load_gather(geom_v, [lane3 + s])
                        gy = plsc.load_gather(geom_v, [lane3 + (s + 1)])
                        gz = plsc.load_gather(geom_v, [lane3 + (s + 2)])
                        ix = ((gx + 50.0) / 0.5).astype(jnp.int32)
                        iy = ((gy + 50.0) / 0.5).astype(jnp.int32)
                        iz = ((gz + 10.0) / 20.0).astype(jnp.int32)
                        ok = ((ix >= 0) & (ix < GRID) & (iy >= 0)
                              & (iy < GRID) & (iz >= 0) & (iz < 1))
                        vox = jnp.where(ok, ix * GRID + iy, R_GRID + tid)
                        idxbuf[l // 8, pl.ds((l % 8) * LANES, LANES)] = vox
                    for j in range(CH // 128):
                        pltpu.sync_copy(xbuf.at[pl.ds(j * 128, 128)],
                                        acc.at[idxbuf.at[j]], add=True)
                return carry
            lax.fori_loop(0, kmax, chunk_body, 0)
            plsc.subcore_barrier()

            rr = tid * ROWS_R
            pltpu.sync_copy(
                acc.at[pl.ds(rr, ROWS_R)],
                out_hbm.at[b, pl.ds(rr, ROWS_R), pl.ds(ch0, CHALF)])
            plsc.subcore_barrier()

    f = pl.kernel(
        body,
        out_type=jax.ShapeDtypeStruct((B, R_GRID, C), jnp.float32),
        mesh=mesh,
        scratch_types=[
            pltpu.VMEM((CH * 3,), jnp.float32),
            pltpu.VMEM((CH, CHALF), jnp.float32),
            pltpu.VMEM((CH // 128, 128), jnp.int32),
            pltpu.VMEM((512, CHALF), jnp.float32),
            pltpu.VMEM_SHARED((R_TOT, CHALF), jnp.float32),
        ],
        compiler_params=pltpu.CompilerParams(
            use_tc_tiling_on_sc=False, needs_layout_passes=False),
    )
    return f(xr, gt)


def _tc_transpose(y):
    B, R, C = y.shape

    def body(in_ref, out_ref):
        out_ref[0] = in_ref[0].T

    return pl.pallas_call(
        body,
        grid=(B,),
        in_specs=[pl.BlockSpec((1, R, C), lambda b: (b, 0, 0))],
        out_specs=pl.BlockSpec((1, C, R), lambda b: (b, 0, 0)),
        out_shape=jax.ShapeDtypeStruct((B, C, R), jnp.float32),
        compiler_params=pltpu.CompilerParams(
            vmem_limit_bytes=100 * 1024 * 1024),
    )(y)


def kernel(x, geometry):
    B, N, D, H, W, C = x.shape
    Np = N * D * H * W
    xr = x.reshape(B, Np, C)
    gr = geometry.reshape(B, Np * 3)
    y = _sc_scatter(xr, gr)
    z = _tc_transpose(y)
    return z.reshape(B, C, GRID, GRID)


# speedup vs baseline: 1.0116x; 1.0116x over previous
import numpy as np

import jax
import jax.numpy as jnp
from jax import lax
from jax.experimental import pallas as pl
from jax.experimental.pallas import tpu as pltpu
from jax.experimental.pallas import tpu_sc as plsc

NC, NS, LANES = 2, 16, 16

GRID = 200
R_GRID = GRID * GRID
R_TOT = R_GRID + NS
ROWS_Z = R_TOT // NS
ROWS_R = R_GRID // NS
CH = 512
CHALF = 32

_SEL = np.zeros((384, 384), dtype=np.float32)
for _t in range(128):
    for _c in range(3):
        _SEL[3 * _t + _c, 128 * _c + _t] = 1.0


def _tc_voxel_ids(g2):
    R3 = g2.shape[0]
    RB = 1080
    assert R3 % RB == 0

    def body(g_ref, s_ref, o_ref):
        m = lax.dot(g_ref[...], s_ref[...], precision=lax.Precision.HIGHEST)
        xs, ys, zs = m[:, :128], m[:, 128:256], m[:, 256:384]
        ix = ((xs + 50.0) / 0.5).astype(jnp.int32)
        iy = ((ys + 50.0) / 0.5).astype(jnp.int32)
        iz = ((zs + 10.0) / 20.0).astype(jnp.int32)
        ok = ((ix >= 0) & (ix < GRID) & (iy >= 0) & (iy < GRID)
              & (iz >= 0) & (iz < 1))
        dump = R_GRID + (lax.broadcasted_iota(jnp.int32, ix.shape, 1) & (NS - 1))
        o_ref[...] = jnp.where(ok, ix * GRID + iy, dump)

    return pl.pallas_call(
        body,
        grid=(R3 // RB,),
        in_specs=[pl.BlockSpec((RB, 384), lambda i: (i, 0)),
                  pl.BlockSpec((384, 384), lambda i: (0, 0))],
        out_specs=pl.BlockSpec((RB, 128), lambda i: (i, 0)),
        out_shape=jax.ShapeDtypeStruct((R3, 128), jnp.int32),
    )(g2, jnp.asarray(_SEL))


def _sc_scatter(xr, vox):
    B, Np, C = xr.shape
    nchunk = vox.shape[1]
    assert Np == nchunk * CH and C == 2 * CHALF
    kmax = (nchunk + NS - 1) // NS
    mesh = plsc.VectorSubcoreMesh(
        core_axis_name="c", subcore_axis_name="s",
        num_cores=NC, num_subcores=NS)

    def body(x_hbm, vox_hbm, out_hbm, xbuf, idxbuf, zb, acc):
        core = lax.axis_index("c")
        tid = lax.axis_index("s")
        ch0 = core * CHALF

        def zb_init(i, carry):
            zb[i, pl.ds(0, LANES)] = jnp.zeros((LANES,), jnp.float32)
            zb[i, pl.ds(LANES, LANES)] = jnp.zeros((LANES,), jnp.float32)
            return carry
        lax.fori_loop(0, zb.shape[0], zb_init, 0)

        for b in range(B):
            r0 = tid * ROWS_Z
            off, rem = 0, ROWS_Z
            while rem > 0:
                n = min(rem, zb.shape[0])
                pltpu.sync_copy(zb.at[pl.ds(0, n)], acc.at[pl.ds(r0 + off, n)])
                off += n
                rem -= n
            plsc.subcore_barrier()

            def chunk_body(k, carry):
                c = k * NS + tid

                @pl.when(c < nchunk)
                def _():
                    base = c * CH
                    pltpu.sync_copy(vox_hbm.at[b, c], idxbuf)
                    pltpu.sync_copy(
                        x_hbm.at[b, pl.ds(base, CH), pl.ds(ch0, CHALF)], xbuf)
                    for j in range(CH // 128):
                        pltpu.sync_copy(xbuf.at[pl.ds(j * 128, 128)],
                                        acc.at[idxbuf.at[j]], add=True)
                return carry
            lax.fori_loop(0, kmax, chunk_body, 0)
            plsc.subcore_barrier()

            rr = tid * ROWS_R
            pltpu.sync_copy(
                acc.at[pl.ds(rr, ROWS_R)],
                out_hbm.at[b, pl.ds(rr, ROWS_R), pl.ds(ch0, CHALF)])
            plsc.subcore_barrier()

    f = pl.kernel(
        body,
        out_type=jax.ShapeDtypeStruct((B, R_GRID, C), jnp.float32),
        mesh=mesh,
        scratch_types=[
            pltpu.VMEM((CH, CHALF), jnp.float32),
            pltpu.VMEM((CH // 128, 128), jnp.int32),
            pltpu.VMEM((512, CHALF), jnp.float32),
            pltpu.VMEM_SHARED((R_TOT, CHALF), jnp.float32),
        ],
        compiler_params=pltpu.CompilerParams(use_tc_tiling_on_sc=False),
    )
    return f(xr, vox)


def _tc_transpose(y):
    B, R, C = y.shape

    def body(in_ref, out_ref):
        out_ref[0] = in_ref[0].T

    return pl.pallas_call(
        body,
        grid=(B,),
        in_specs=[pl.BlockSpec((1, R, C), lambda b: (b, 0, 0))],
        out_specs=pl.BlockSpec((1, C, R), lambda b: (b, 0, 0)),
        out_shape=jax.ShapeDtypeStruct((B, C, R), jnp.float32),
        compiler_params=pltpu.CompilerParams(
            vmem_limit_bytes=100 * 1024 * 1024),
    )(y)


def kernel(x, geometry):
    B, N, D, H, W, C = x.shape
    Np = N * D * H * W
    xr = x.reshape(B, Np, C)
    g2 = geometry.reshape(B * Np * 3 // 384, 384)
    vox = _tc_voxel_ids(g2).reshape(B, Np // CH, CH // 128, 128)
    y = _sc_scatter(xr, vox)
    z = _tc_transpose(y)
    return z.reshape(B, C, GRID, GRID)


# speedup vs baseline: 5.4677x; 5.4052x over previous
import jax
import jax.numpy as jnp
from jax import lax
from jax.experimental import pallas as pl
from jax.experimental.pallas import tpu as pltpu
from jax.experimental.pallas import tpu_sc as plsc

NC, NS, LANES = 2, 16, 16

GRID = 200
R_GRID = GRID * GRID
R_TOT = R_GRID + NS
ROWS_Z = R_TOT // NS
ROWS_R = R_GRID // NS
CH = 512
CHALF = 32

def _tc_voxel_ids(gt, D, W):
    R3 = gt.shape[0]
    RB = 42
    assert R3 % RB == 0

    def body(g_ref, o_ref):
        g = g_ref[...]
        xs, ys, zs = g[:, 0], g[:, 1], g[:, 2]
        ix = ((xs + 50.0) / 0.5).astype(jnp.int32)
        iy = ((ys + 50.0) / 0.5).astype(jnp.int32)
        iz = ((zs + 10.0) / 20.0).astype(jnp.int32)
        ok = ((ix >= 0) & (ix < GRID) & (iy >= 0) & (iy < GRID)
              & (iz >= 0) & (iz < 1))
        dump = R_GRID + (lax.broadcasted_iota(jnp.int32, ix.shape, 2) & (NS - 1))
        vox = jnp.where(ok, ix * GRID + iy, dump)
        o_ref[...] = jnp.swapaxes(vox, 1, 2)

    return pl.pallas_call(
        body,
        grid=(R3 // RB,),
        in_specs=[pl.BlockSpec((RB, 3, D, W), lambda i: (i, 0, 0, 0))],
        out_specs=pl.BlockSpec((RB, W, D), lambda i: (i, 0, 0)),
        out_shape=jax.ShapeDtypeStruct((R3, W, D), jnp.int32),
    )(gt)


def _sc_scatter(xr, vox):
    B, Np, C = xr.shape
    nchunk = vox.shape[1]
    assert Np == nchunk * CH and C == 2 * CHALF
    kmax = (nchunk + NS - 1) // NS
    mesh = plsc.VectorSubcoreMesh(
        core_axis_name="c", subcore_axis_name="s",
        num_cores=NC, num_subcores=NS)

    def body(x_hbm, vox_hbm, out_hbm, xbuf, idxbuf, zb, acc):
        core = lax.axis_index("c")
        tid = lax.axis_index("s")
        ch0 = core * CHALF

        def zb_init(i, carry):
            zb[i, pl.ds(0, LANES)] = jnp.zeros((LANES,), jnp.float32)
            zb[i, pl.ds(LANES, LANES)] = jnp.zeros((LANES,), jnp.float32)
            return carry
        lax.fori_loop(0, zb.shape[0], zb_init, 0)

        for b in range(B):
            r0 = tid * ROWS_Z
            off, rem = 0, ROWS_Z
            while rem > 0:
                n = min(rem, zb.shape[0])
                pltpu.sync_copy(zb.at[pl.ds(0, n)], acc.at[pl.ds(r0 + off, n)])
                off += n
                rem -= n
            plsc.subcore_barrier()

            def chunk_body(k, carry):
                c = k * NS + tid

                @pl.when(c < nchunk)
                def _():
                    base = c * CH
                    pltpu.sync_copy(vox_hbm.at[b, c], idxbuf)
                    pltpu.sync_copy(
                        x_hbm.at[b, pl.ds(base, CH), pl.ds(ch0, CHALF)], xbuf)
                    for j in range(CH // 128):
                        pltpu.sync_copy(xbuf.at[pl.ds(j * 128, 128)],
                                        acc.at[idxbuf.at[j]], add=True)
                return carry
            lax.fori_loop(0, kmax, chunk_body, 0)
            plsc.subcore_barrier()

            rr = tid * ROWS_R
            pltpu.sync_copy(
                acc.at[pl.ds(rr, ROWS_R)],
                out_hbm.at[b, pl.ds(rr, ROWS_R), pl.ds(ch0, CHALF)])
            plsc.subcore_barrier()

    f = pl.kernel(
        body,
        out_type=jax.ShapeDtypeStruct((B, R_GRID, C), jnp.float32),
        mesh=mesh,
        scratch_types=[
            pltpu.VMEM((CH, CHALF), jnp.float32),
            pltpu.VMEM((CH // 128, 128), jnp.int32),
            pltpu.VMEM((512, CHALF), jnp.float32),
            pltpu.VMEM_SHARED((R_TOT, CHALF), jnp.float32),
        ],
        compiler_params=pltpu.CompilerParams(use_tc_tiling_on_sc=False),
    )
    return f(xr, vox)


def _tc_transpose(y):
    B, R, C = y.shape

    def body(in_ref, out_ref):
        out_ref[0] = in_ref[0].T

    return pl.pallas_call(
        body,
        grid=(B,),
        in_specs=[pl.BlockSpec((1, R, C), lambda b: (b, 0, 0))],
        out_specs=pl.BlockSpec((1, C, R), lambda b: (b, 0, 0)),
        out_shape=jax.ShapeDtypeStruct((B, C, R), jnp.float32),
        compiler_params=pltpu.CompilerParams(
            vmem_limit_bytes=100 * 1024 * 1024),
    )(y)


def kernel(x, geometry):
    B, N, D, H, W, C = x.shape
    Np = N * D * H * W
    xr = x.transpose(0, 1, 3, 4, 2, 5).reshape(B, Np, C)
    gt = geometry.transpose(0, 1, 3, 5, 2, 4).reshape(B * N * H, 3, D, W)
    vox = _tc_voxel_ids(gt, D, W).reshape(B, Np // CH, CH // 128, 128)
    y = _sc_scatter(xr, vox)
    z = _tc_transpose(y)
    return z.reshape(B, C, GRID, GRID)
